# baseline (speedup 1.0000x reference)
import functools

import jax
import jax.numpy as jnp
from jax import lax
from jax.experimental import pallas as pl
from jax.experimental.pallas import tpu as pltpu
from jax.experimental.pallas import tpu_sc as plsc


def kernel(input_ids, embedding_weight):
    B, L = input_ids.shape
    V, D = embedding_weight.shape
    N = B * L

    info = plsc.get_sparse_core_info()
    NC, NS, NL = info.num_cores, info.num_subcores, info.num_lanes
    NW = NC * NS
    BBLK = B // NW

    ids_t = input_ids.T
    mesh = plsc.VectorSubcoreMesh(core_axis_name="c", subcore_axis_name="s")

    @functools.partial(
        pl.kernel,
        mesh=mesh,
        out_type=jax.ShapeDtypeStruct((N // 32, 8, 128), jnp.float32),
        scratch_types=[
            pltpu.VMEM((L, BBLK), jnp.int32),
            *[pltpu.VMEM((BBLK, D), jnp.float32) for _ in range(2)],
            *[pltpu.VMEM((D, BBLK), jnp.float32) for _ in range(2)],
            *[pltpu.SemaphoreType.DMA for _ in range(2)],
            *[pltpu.SemaphoreType.DMA for _ in range(2)],
        ],
        compiler_params=pltpu.CompilerParams(
            use_tc_tiling_on_sc=False, needs_layout_passes=False
        ),
    )
    def emb(ids_hbm, table_hbm, out_hbm, idx_v, *scr):
        bufs, tbufs, gsems, ssems = (
            scr[0:2],
            scr[2:4],
            scr[4:6],
            scr[6:8],
        )
        w = lax.axis_index("s") * NC + lax.axis_index("c")

        pltpu.sync_copy(ids_hbm.at[:, pl.ds(w * BBLK, BBLK)], idx_v)
        iota = lax.iota(jnp.int32, NL)

        def gather(g, k):
            pltpu.async_copy(table_hbm.at[idx_v.at[g]], bufs[k], gsems[k])

        def gwait(k):
            pltpu.make_async_copy(
                table_hbm.at[idx_v.at[0]], bufs[k], gsems[k]
            ).wait()

        def process(k):
            buf, tbuf = bufs[k], tbufs[k]
            for b0 in range(0, BBLK, NL):
                rows = iota + b0
                for d in range(D):
                    tbuf[d, pl.ds(b0, NL)] = plsc.load_gather(
                        buf, [rows, jnp.full((NL,), d, jnp.int32)]
                    )

        def store(g, k):
            for tr in range(4):
                pltpu.async_copy(
                    tbufs[k].at[pl.ds(tr * 8, 8)],
                    out_hbm.at[(g * 4 + tr) * 32 + w],
                    ssems[k],
                )

        def swait(k):
            for tr in range(4):
                pltpu.make_async_copy(
                    tbufs[k].at[pl.ds(tr * 8, 8)],
                    out_hbm.at[0],
                    ssems[k],
                ).wait()

        for g in range(2):
            gather(g, g)
        for g in range(2):
            gwait(g)
            process(g)
            gather(g + 2, g)
            store(g, g)

        def body(p, carry):
            for h in range(2):
                g = p * 2 + h
                gwait(h)
                swait(h)
                process(h)
                gather(g + 2, h)
                store(g, h)
            return carry

        lax.fori_loop(1, L // 2 - 1, body, 0)

        for g in range(L - 2, L):
            h = g % 2
            gwait(h)
            swait(h)
            process(h)
            store(g, h)
        for h in range(2):
            swait(h)

    out3 = emb(ids_t, embedding_weight)
    out5 = out3.reshape(L, 4, 32, 8, 128)
    return out5.transpose(2, 4, 0, 1, 3).reshape(B, L, D)


# speedup vs baseline: 1.6694x; 1.6694x over previous
import functools

import jax
import jax.numpy as jnp
from jax import lax
from jax.experimental import pallas as pl
from jax.experimental.pallas import tpu as pltpu
from jax.experimental.pallas import tpu_sc as plsc

GC = 640
K = 4


def kernel(input_ids, embedding_weight):
    B, L = input_ids.shape
    V, D = embedding_weight.shape
    N = B * L

    info = plsc.get_sparse_core_info()
    NC, NS = info.num_cores, info.num_subcores
    NW = NC * NS
    assert N % (NW * GC * K) == 0
    n_per_w = N // NW
    ngrp = n_per_w // GC
    nblk = ngrp // K

    ids3 = input_ids.reshape(NW, n_per_w).astype(jnp.int32)
    mesh = plsc.VectorSubcoreMesh(core_axis_name="c", subcore_axis_name="s")

    @functools.partial(
        pl.kernel,
        mesh=mesh,
        out_type=jax.ShapeDtypeStruct((N, 128), jnp.float32),
        scratch_types=[
            pltpu.VMEM((n_per_w,), jnp.int32),
            *[pltpu.VMEM((GC, D), jnp.float32) for _ in range(K)],
            *[pltpu.SemaphoreType.DMA for _ in range(K)],
        ],
        compiler_params=pltpu.CompilerParams(use_tc_tiling_on_sc=False),
    )
    def emb(ids_hbm, table_hbm, out_hbm, idx_v, *bufsem):
        bufs, sems = bufsem[:K], bufsem[K:]
        wid = lax.axis_index("s") * NC + lax.axis_index("c")
        base = wid * n_per_w
        pltpu.sync_copy(ids_hbm.at[wid], idx_v)

        def gather(c, k):
            pltpu.async_copy(
                table_hbm.at[idx_v.at[pl.ds(c * GC, GC)]], bufs[k], sems[k]
            )

        def gwait(k):
            pltpu.make_async_copy(
                table_hbm.at[idx_v.at[pl.ds(0, GC)]], bufs[k], sems[k]
            ).wait()

        def store(c, k):
            pltpu.sync_copy(
                bufs[k], out_hbm.at[pl.ds(base + c * GC, GC), pl.ds(0, D)]
            )

        for k in range(K):
            gather(k, k)

        def body(blk, carry):
            for k in range(K):
                c = blk * K + k
                gwait(k)
                store(c, k)
                gather(c + K, k)
            return carry

        lax.fori_loop(0, nblk - 1, body, 0)
        for k in range(K):
            c = (nblk - 1) * K + k
            gwait(k)
            store(c, k)

    out = emb(ids3, embedding_weight)
    return out[:, :D].reshape(B, L, D)
